# Initial kernel scaffold; baseline (speedup 1.0000x reference)
import jax
import jax.numpy as jnp
from jax.experimental import pallas as pl


def kernel(x, edge_index, edge_attr, batch, xe1, xe2, ee1, ee2, W1, b1, W2, b2, gamma, beta, Wp, bp):
    raise NotImplementedError("write your pallas kernel here")


# speedup vs baseline: 1.2353x; 1.2353x over previous
import functools

import jax
import jax.numpy as jnp
from jax import lax
from jax.experimental import pallas as pl
from jax.experimental.pallas import tpu as pltpu
from jax.experimental.pallas import tpu_sc as plsc

N = 10000
EMB = 256
DH = 128
G = 512
E = 170000
CHUNK = 128
EPT = 10752
TCH = 84 + 16
EPADT = 15 * EPT + TCH * CHUNK
NROW_PAD = 10240
TRASH = NROW_PAD - 1
DPT = N // 16
ZROWS = NROW_PAD // 16
OSTEP = 624
OROWS = 640

RB = 2000
NB = N // RB

_PREC = lax.Precision.DEFAULT
_PREC_X = lax.Precision.HIGHEST
_f32 = jnp.float32

_sc_mesh = plsc.VectorSubcoreMesh(core_axis_name="c", subcore_axis_name="s")


def _sc_embed_body(t0_hbm, idx_hbm, out_hbm, idxv, idx2, rows, sem):
    c = lax.axis_index("c")
    s = lax.axis_index("s")
    off = c * 360
    base = s * ZROWS

    def body(k, carry):
        eb = base + k * CHUNK
        pltpu.sync_copy(idx_hbm.at[pl.ds(eb, CHUNK)], idxv)
        for i in range(CHUNK // 16):
            sl = pl.ds(i * 16, 16)
            idx2[sl] = idxv[sl] + off
        pltpu.async_copy(t0_hbm.at[idx2], rows, sem).wait()
        pltpu.sync_copy(rows, out_hbm.at[pl.ds(c * NROW_PAD + eb, CHUNK)])
        return carry

    lax.fori_loop(0, ZROWS // CHUNK, body, 0)


@functools.partial(
    pl.kernel,
    out_type=jax.ShapeDtypeStruct((2 * NROW_PAD, DH), _f32),
    mesh=_sc_mesh,
    scratch_types=[
        pltpu.VMEM((CHUNK,), jnp.int32),
        pltpu.VMEM((CHUNK,), jnp.int32),
        pltpu.VMEM((CHUNK, DH), _f32),
        pltpu.SemaphoreType.DMA,
    ],
)
def _sc_embed(t0_hbm, idx_hbm, out_hbm, idxv, idx2, rows, sem):
    _sc_embed_body(t0_hbm, idx_hbm, out_hbm, idxv, idx2, rows, sem)


def _sc_spmm_body(h2_hbm, t18_hbm, src_hbm, c18_hbm, dst_hbm, own_hbm,
                  zeros_hbm, out_hbm,
                  srcv, idx2, c18v, idxT, dstv, ownv, dstm, rows, erows,
                  sem, acc):
    c = lax.axis_index("c")
    s = lax.axis_index("s")
    pltpu.sync_copy(zeros_hbm, acc.at[pl.ds(s * ZROWS, ZROWS)])
    plsc.subcore_barrier()
    base = s * EPT
    off = c * N
    offT = c * 18

    def body(k, carry):
        eb = base + k * CHUNK
        pltpu.sync_copy(src_hbm.at[pl.ds(eb, CHUNK)], srcv)
        pltpu.sync_copy(c18_hbm.at[pl.ds(eb, CHUNK)], c18v)
        pltpu.sync_copy(dst_hbm.at[pl.ds(eb, CHUNK)], dstv)
        pltpu.sync_copy(own_hbm.at[pl.ds(eb, CHUNK)], ownv)
        for i in range(CHUNK // 16):
            sl = pl.ds(i * 16, 16)
            idx2[sl] = srcv[sl] + off
            idxT[sl] = c18v[sl] + offT
            dstm[sl] = jnp.where(ownv[sl] == s, dstv[sl], TRASH)
        pltpu.async_copy(h2_hbm.at[idx2], rows, sem).wait()
        pltpu.async_copy(t18_hbm.at[idxT], erows, sem).wait()

        def radd(r, carry2):
            for i in range(DH // 16):
                sl = pl.ds(i * 16, 16)
                rows[r, sl] = rows[r, sl] + erows[r, sl]
            return carry2

        lax.fori_loop(0, CHUNK, radd, 0)
        pltpu.sync_copy(rows, acc.at[dstm], add=True)
        return carry

    lax.fori_loop(0, TCH, body, 0)
    plsc.subcore_barrier()
    pltpu.sync_copy(acc.at[pl.ds(s * OSTEP, OROWS)],
                    out_hbm.at[pl.ds(c * N + s * OSTEP, OROWS)])


@functools.partial(
    pl.kernel,
    out_type=jax.ShapeDtypeStruct((2 * N, DH), _f32),
    mesh=_sc_mesh,
    scratch_types=[
        pltpu.VMEM((CHUNK,), jnp.int32),
        pltpu.VMEM((CHUNK,), jnp.int32),
        pltpu.VMEM((CHUNK,), jnp.int32),
        pltpu.VMEM((CHUNK,), jnp.int32),
        pltpu.VMEM((CHUNK,), jnp.int32),
        pltpu.VMEM((CHUNK,), jnp.int32),
        pltpu.VMEM((CHUNK,), jnp.int32),
        pltpu.VMEM((CHUNK, DH), _f32),
        pltpu.VMEM((CHUNK, DH), _f32),
        pltpu.SemaphoreType.DMA,
        pltpu.VMEM_SHARED((NROW_PAD, DH), _f32),
    ],
)
def _sc_spmm(h2_hbm, t18_hbm, src_hbm, c18_hbm, dst_hbm, own_hbm,
             zeros_hbm, out_hbm,
             srcv, idx2, c18v, idxT, dstv, ownv, dstm, rows, erows,
             sem, acc):
    _sc_spmm_body(h2_hbm, t18_hbm, src_hbm, c18_hbm, dst_hbm, own_hbm,
                  zeros_hbm, out_hbm,
                  srcv, idx2, c18v, idxT, dstv, ownv, dstm, rows, erows,
                  sem, acc)


def _mlp_a_body(agg2_ref, W1_ref, b1_ref, W2_ref, b2_ref, o_ref, sums_ref):
    i = pl.program_id(0)
    full = jnp.concatenate([agg2_ref[0], agg2_ref[1]], axis=1)
    hid = jnp.maximum(
        jnp.dot(full, W1_ref[:], preferred_element_type=_f32,
                precision=_PREC) + b1_ref[:], 0.0)
    o = jnp.dot(hid, W2_ref[:], preferred_element_type=_f32,
                precision=_PREC) + b2_ref[:]
    o_ref[:] = o
    ps = jnp.sum(o, axis=0, keepdims=True)

    @pl.when(i == 0)
    def _():
        sums_ref[0:1, :] = ps

    @pl.when(i > 0)
    def _():
        sums_ref[0:1, :] += ps


_mlp_a = pl.pallas_call(
    _mlp_a_body,
    grid=(NB,),
    in_specs=[
        pl.BlockSpec((2, RB, DH), lambda i: (0, i, 0)),
        pl.BlockSpec((EMB, 2 * EMB), lambda i: (0, 0)),
        pl.BlockSpec((1, 2 * EMB), lambda i: (0, 0)),
        pl.BlockSpec((2 * EMB, EMB), lambda i: (0, 0)),
        pl.BlockSpec((1, EMB), lambda i: (0, 0)),
    ],
    out_specs=(
        pl.BlockSpec((RB, EMB), lambda i: (i, 0)),
        pl.BlockSpec((8, EMB), lambda i: (0, 0)),
    ),
    out_shape=(
        jax.ShapeDtypeStruct((N, EMB), _f32),
        jax.ShapeDtypeStruct((8, EMB), _f32),
    ),
)


def _bnvar_body(o_ref, sums_ref, var_ref):
    i = pl.program_id(0)
    d = o_ref[:] - sums_ref[0:1, :] / N
    pv = jnp.sum(d * d, axis=0, keepdims=True)

    @pl.when(i == 0)
    def _():
        var_ref[0:1, :] = pv

    @pl.when(i > 0)
    def _():
        var_ref[0:1, :] += pv


_bnvar = pl.pallas_call(
    _bnvar_body,
    grid=(NB,),
    in_specs=[
        pl.BlockSpec((RB, EMB), lambda i: (i, 0)),
        pl.BlockSpec((8, EMB), lambda i: (0, 0)),
    ],
    out_specs=pl.BlockSpec((8, EMB), lambda i: (0, 0)),
    out_shape=jax.ShapeDtypeStruct((8, EMB), _f32),
)


def _mlp_b_body(emit_split, emit_raw, o_ref, sums_ref, var_ref, g_ref,
                be_ref, *out_refs):
    o = o_ref[:]
    m = sums_ref[0:1, :] / N
    v = var_ref[0:1, :] / N
    bn = (o - m) * lax.rsqrt(v + 1e-5) * g_ref[:] + be_ref[:]
    i = 0
    if emit_split:
        r = jnp.maximum(bn, 0.0)
        out_refs[i][0] = r[:, :DH]
        out_refs[i][1] = r[:, DH:]
        i += 1
    if emit_raw:
        out_refs[i][:] = bn


def _make_mlp_b(emit_split, emit_raw):
    specs, shapes = [], []
    if emit_split:
        specs.append(pl.BlockSpec((2, RB, DH), lambda i: (0, i, 0)))
        shapes.append(jax.ShapeDtypeStruct((2, N, DH), _f32))
    if emit_raw:
        specs.append(pl.BlockSpec((RB, EMB), lambda i: (i, 0)))
        shapes.append(jax.ShapeDtypeStruct((N, EMB), _f32))
    return pl.pallas_call(
        functools.partial(_mlp_b_body, emit_split, emit_raw),
        grid=(NB,),
        in_specs=[
            pl.BlockSpec((RB, EMB), lambda i: (i, 0)),
            pl.BlockSpec((8, EMB), lambda i: (0, 0)),
            pl.BlockSpec((8, EMB), lambda i: (0, 0)),
            pl.BlockSpec((1, EMB), lambda i: (0, 0)),
            pl.BlockSpec((1, EMB), lambda i: (0, 0)),
        ],
        out_specs=tuple(specs) if len(specs) > 1 else specs[0],
        out_shape=tuple(shapes) if len(shapes) > 1 else shapes[0],
    )


_mlp_b_split = _make_mlp_b(True, False)
_mlp_b_both = _make_mlp_b(True, True)
_mlp_b_raw = _make_mlp_b(False, True)


def _pool_body(bt_ref, hn_ref, hs_ref, Wp_ref, bp_ref, out_ref,
               pn_acc, ps_acc, cnt_acc):
    i = pl.program_id(0)
    gi = lax.broadcasted_iota(jnp.int32, (RB, G), 1)
    B = (bt_ref[:] == gi).astype(_f32)
    pn = lax.dot_general(B, hn_ref[:], (((0,), (0,)), ((), ())),
                         preferred_element_type=_f32, precision=_PREC_X)
    ps = lax.dot_general(B, hs_ref[:], (((0,), (0,)), ((), ())),
                         preferred_element_type=_f32, precision=_PREC_X)
    cn = jnp.sum(B, axis=0)[:, None]

    @pl.when(i == 0)
    def _():
        pn_acc[:] = pn
        ps_acc[:] = ps
        cnt_acc[:] = cn

    @pl.when(i > 0)
    def _():
        pn_acc[:] += pn
        ps_acc[:] += ps
        cnt_acc[:] += cn

    @pl.when(i == NB - 1)
    def _():
        counts = jnp.maximum(cnt_acc[:], 1.0)
        rep = jnp.concatenate([pn_acc[:] / counts, ps_acc[:] / counts],
                              axis=1)
        out_ref[:] = jnp.dot(rep, Wp_ref[:], preferred_element_type=_f32,
                             precision=_PREC_X) + bp_ref[:]


_pool = pl.pallas_call(
    _pool_body,
    grid=(NB,),
    in_specs=[
        pl.BlockSpec((RB, 1), lambda i: (i, 0)),
        pl.BlockSpec((RB, EMB), lambda i: (i, 0)),
        pl.BlockSpec((RB, EMB), lambda i: (i, 0)),
        pl.BlockSpec((2 * EMB, 10), lambda i: (0, 0)),
        pl.BlockSpec((1, 10), lambda i: (0, 0)),
    ],
    out_specs=pl.BlockSpec((G, 10), lambda i: (0, 0)),
    out_shape=jax.ShapeDtypeStruct((G, 10), _f32),
    scratch_shapes=[
        pltpu.VMEM((G, EMB), _f32),
        pltpu.VMEM((G, EMB), _f32),
        pltpu.VMEM((G, 1), _f32),
    ],
)


def kernel(x, edge_index, edge_attr, batch, xe1, xe2, ee1, ee2,
           W1, b1, W2, b2, gamma, beta, Wp, bp):
    loop_idx = jnp.arange(N, dtype=jnp.int32)
    src_all = jnp.concatenate([edge_index[0], loop_idx])
    dst_all = jnp.concatenate([edge_index[1], loop_idx])
    c18_all = jnp.concatenate([edge_attr[:, 0] * 3 + edge_attr[:, 1],
                               jnp.full((N,), 12, jnp.int32)])
    perm = jnp.argsort(dst_all, stable=True)
    srcs = src_all[perm]
    dsts = dst_all[perm]
    c18s = c18_all[perm]
    padn = EPADT - E
    first_e = jnp.searchsorted(dsts, jnp.arange(N, dtype=jnp.int32),
                               side="left").astype(jnp.int32)
    owner_d = first_e // EPT
    own = owner_d[dsts]
    srcs = jnp.concatenate([srcs, jnp.zeros((padn,), jnp.int32)])
    c18s = jnp.concatenate([c18s, jnp.zeros((padn,), jnp.int32)])
    dsts_p = jnp.concatenate([dsts, jnp.zeros((padn,), jnp.int32)])
    own_p = jnp.concatenate([own, jnp.full((padn,), 16, jnp.int32)])
    zeros = jnp.zeros((ZROWS, DH), _f32)

    t0 = (xe1[:, None, :] + xe2[None, :, :]).reshape(360, EMB)
    t0c = jnp.concatenate([t0[:, :DH], t0[:, DH:]], axis=0)
    idx0 = x[:, 0] * 3 + x[:, 1]
    idx0p = jnp.concatenate([idx0, jnp.zeros((NROW_PAD - N,), jnp.int32)])
    h2p = _sc_embed(t0c, idx0p)
    h2 = h2p.reshape(2, NROW_PAD, DH)[:, :N].reshape(2 * N, DH)

    h_node = None
    h_struct = None
    for l in range(6):
        t18 = (ee1[l][:, None, :] + ee2[l][None, :, :]).reshape(18, EMB)
        t18c = jnp.concatenate([t18[:, :DH], t18[:, DH:]], axis=0)
        agg = _sc_spmm(h2, t18c, srcs, c18s, dsts_p, own_p, zeros)
        agg2 = agg.reshape(2, N, DH)
        o, sums = _mlp_a(agg2, W1[l], b1[l][None], W2[l], b2[l][None])
        var = _bnvar(o, sums)
        bargs = (o, sums, var, gamma[l][None], beta[l][None])
        if l == 2:
            hsp, h_node = _mlp_b_both(*bargs)
        elif l == 5:
            h_struct = _mlp_b_raw(*bargs)
            hsp = None
        else:
            hsp = _mlp_b_split(*bargs)
        if hsp is not None:
            h2 = hsp.reshape(2 * N, DH)

    return _pool(batch[:, None].astype(jnp.int32), h_node, h_struct, Wp,
                 bp[None])
